# Initial kernel scaffold; baseline (speedup 1.0000x reference)
import jax
import jax.numpy as jnp
from jax.experimental import pallas as pl


def kernel(inputs):
    raise NotImplementedError("write your pallas kernel here")


# speedup vs baseline: 1.7481x; 1.7481x over previous
import functools

import jax
import jax.numpy as jnp
from jax import lax
from jax.experimental import pallas as pl
from jax.experimental.pallas import tpu as pltpu
from jax.experimental.pallas import tpu_sc as plsc

B, L, F = 16, 2048, 512
ROWS = B * L

_LBLK = 128


def _mask_body(x_ref, o_ref):
    nz = (x_ref[...] != 0.0).astype(jnp.float32)
    s = jnp.sum(jnp.sum(nz, axis=2), axis=0, keepdims=True)
    o_ref[...] = (s > 0.0).astype(jnp.int32)


_colmask_call = pl.pallas_call(
    _mask_body,
    grid=(L // _LBLK,),
    in_specs=[pl.BlockSpec((B, _LBLK, F), lambda i: (0, i, 0))],
    out_specs=pl.BlockSpec((1, _LBLK), lambda i: (0, i)),
    out_shape=jax.ShapeDtypeStruct((1, L), jnp.int32),
)


def _dest_body(m_ref, o_ref):
    kept = m_ref[...]
    r = lax.broadcasted_iota(jnp.int32, (L, L), 0)
    c = lax.broadcasted_iota(jnp.int32, (L, L), 1)
    tri = (r <= c).astype(jnp.bfloat16)
    incl = lax.dot_general(
        kept.astype(jnp.bfloat16), tri,
        (((1,), (0,)), ((), ())),
        preferred_element_type=jnp.float32,
    ).astype(jnp.int32)
    total = jnp.sum(kept)
    pe = incl - kept
    col = lax.broadcasted_iota(jnp.int32, (1, L), 1)
    dest = jnp.where(kept > 0, pe, total + col - pe)
    row = lax.broadcasted_iota(jnp.int32, (B, L), 0)
    o_ref[...] = dest + row * L


_dest_call = pl.pallas_call(
    _dest_body,
    out_shape=jax.ShapeDtypeStruct((B, L), jnp.int32),
)


_CHUNK = 64


@functools.cache
def _make_scatter():
    info = plsc.get_sparse_core_info()
    nc, ns = info.num_cores, info.num_subcores
    nw = nc * ns
    rpw = ROWS // nw
    mesh = plsc.VectorSubcoreMesh(core_axis_name="c", subcore_axis_name="s")

    @functools.partial(
        pl.kernel,
        mesh=mesh,
        out_type=jax.ShapeDtypeStruct((ROWS, F), jnp.float32),
        scratch_types=[
            pltpu.VMEM((_CHUNK,), jnp.int32),
            pltpu.VMEM((_CHUNK, F), jnp.float32),
            pltpu.SemaphoreType.DMA,
        ],
    )
    def scatter(rows_hbm, idx_hbm, out_hbm, idx_v, rows_v, sem):
        wid = lax.axis_index("s") * nc + lax.axis_index("c")
        base = wid * rpw

        def body(i, carry):
            off = base + i * _CHUNK
            pltpu.sync_copy(rows_hbm.at[pl.ds(off, _CHUNK)], rows_v)
            pltpu.sync_copy(idx_hbm.at[pl.ds(off, _CHUNK)], idx_v)
            pltpu.async_copy(rows_v, out_hbm.at[idx_v], sem).wait()
            return carry

        lax.fori_loop(0, rpw // _CHUNK, body, 0)

    return scatter


def kernel(inputs):
    colmask = _colmask_call(inputs)
    dest = _dest_call(colmask)
    out = _make_scatter()(inputs.reshape(ROWS, F), dest.reshape(ROWS))
    return out.reshape(B, L, F)


# speedup vs baseline: 1.9359x; 1.1074x over previous
import functools

import jax
import jax.numpy as jnp
from jax import lax
from jax.experimental import pallas as pl
from jax.experimental.pallas import tpu as pltpu
from jax.experimental.pallas import tpu_sc as plsc

B, L, F = 16, 2048, 512
ROWS = B * L

_LBLK = 128


def _mask_body(x_ref, o_ref):
    nz = (x_ref[...] != 0.0).astype(jnp.float32)
    s = jnp.sum(jnp.sum(nz, axis=2), axis=0, keepdims=True)
    o_ref[...] = (s > 0.0).astype(jnp.int32)


_colmask_call = pl.pallas_call(
    _mask_body,
    grid=(L // _LBLK,),
    in_specs=[pl.BlockSpec((B, _LBLK, F), lambda i: (0, i, 0))],
    out_specs=pl.BlockSpec((1, _LBLK), lambda i: (0, i)),
    out_shape=jax.ShapeDtypeStruct((1, L), jnp.int32),
)


def _dest_body(m_ref, o_ref):
    kept = m_ref[...]
    r = lax.broadcasted_iota(jnp.int32, (L, L), 0)
    c = lax.broadcasted_iota(jnp.int32, (L, L), 1)
    tri = (r <= c).astype(jnp.bfloat16)
    incl = lax.dot_general(
        kept.astype(jnp.bfloat16), tri,
        (((1,), (0,)), ((), ())),
        preferred_element_type=jnp.float32,
    ).astype(jnp.int32)
    total = jnp.sum(kept)
    pe = incl - kept
    col = lax.broadcasted_iota(jnp.int32, (1, L), 1)
    dest = jnp.where(kept > 0, pe, total + col - pe)
    row = lax.broadcasted_iota(jnp.int32, (B, L), 0)
    o_ref[...] = dest + row * L


_dest_call = pl.pallas_call(
    _dest_body,
    out_shape=jax.ShapeDtypeStruct((B, L), jnp.int32),
)


_CHUNK = 64


@functools.cache
def _make_scatter():
    info = plsc.get_sparse_core_info()
    nc, ns = info.num_cores, info.num_subcores
    nw = nc * ns
    rpw = ROWS // nw
    nchunks = rpw // _CHUNK
    mesh = plsc.VectorSubcoreMesh(core_axis_name="c", subcore_axis_name="s")

    @functools.partial(
        pl.kernel,
        mesh=mesh,
        out_type=jax.ShapeDtypeStruct((ROWS, F), jnp.float32),
        scratch_types=[
            pltpu.VMEM((nchunks, _CHUNK), jnp.int32),
            pltpu.VMEM((_CHUNK, F), jnp.float32),
            pltpu.VMEM((_CHUNK, F), jnp.float32),
            pltpu.SemaphoreType.DMA,
            pltpu.SemaphoreType.DMA,
        ],
    )
    def scatter(rows_hbm, idx_hbm, out_hbm, idx_v, rows_a, rows_b, sem_a,
                sem_b):
        wid = lax.axis_index("s") * nc + lax.axis_index("c")
        base = wid * rpw
        pltpu.sync_copy(idx_hbm.at[pl.ds(wid * nchunks, nchunks)], idx_v)

        bufs = (rows_a, rows_b)
        sems = (sem_a, sem_b)
        pending = [None, None]
        for j in range(nchunks):
            b = j & 1
            if pending[b] is not None:
                pending[b].wait()
            pltpu.sync_copy(rows_hbm.at[pl.ds(base + j * _CHUNK, _CHUNK)],
                            bufs[b])
            pending[b] = pltpu.async_copy(bufs[b], out_hbm.at[idx_v.at[j]],
                                          sems[b])
        pending[0].wait()
        pending[1].wait()

    return scatter


def kernel(inputs):
    colmask = _colmask_call(inputs)
    dest = _dest_call(colmask)
    out = _make_scatter()(inputs.reshape(ROWS, F),
                          dest.reshape(ROWS // _CHUNK, _CHUNK))
    return out.reshape(B, L, F)


# speedup vs baseline: 2.5272x; 1.3055x over previous
import functools

import jax
import jax.numpy as jnp
from jax import lax
from jax.experimental import pallas as pl
from jax.experimental.pallas import tpu as pltpu
from jax.experimental.pallas import tpu_sc as plsc

B, L, F = 16, 2048, 512
ROWS = B * L

_LBLK = 128


def _mask_body(x_ref, o_ref):
    nz = (x_ref[...] != 0.0).astype(jnp.float32)
    s = jnp.sum(jnp.sum(nz, axis=2), axis=0, keepdims=True)
    o_ref[...] = (s > 0.0).astype(jnp.int32)


_colmask_call = pl.pallas_call(
    _mask_body,
    grid=(L // _LBLK,),
    in_specs=[pl.BlockSpec((B, _LBLK, F), lambda i: (0, i, 0))],
    out_specs=pl.BlockSpec((1, _LBLK), lambda i: (0, i)),
    out_shape=jax.ShapeDtypeStruct((1, L), jnp.int32),
)


def _dest_body(m_ref, o_ref):
    kept = m_ref[...]
    r = lax.broadcasted_iota(jnp.int32, (L, L), 0)
    c = lax.broadcasted_iota(jnp.int32, (L, L), 1)
    tri = (r <= c).astype(jnp.bfloat16)
    incl = lax.dot_general(
        kept.astype(jnp.bfloat16), tri,
        (((1,), (0,)), ((), ())),
        preferred_element_type=jnp.float32,
    ).astype(jnp.int32)
    total = jnp.sum(kept)
    pe = incl - kept
    col = lax.broadcasted_iota(jnp.int32, (1, L), 1)
    dest = jnp.where(kept > 0, pe, total + col - pe)
    row = lax.broadcasted_iota(jnp.int32, (B, L), 0)
    o_ref[...] = dest + row * L


_dest_call = pl.pallas_call(
    _dest_body,
    out_shape=jax.ShapeDtypeStruct((B, L), jnp.int32),
)


_CHUNK = 64


@functools.cache
def _make_scatter():
    info = plsc.get_sparse_core_info()
    nc, ns = info.num_cores, info.num_subcores
    nw = nc * ns
    rpw = ROWS // nw
    nchunks = rpw // _CHUNK
    mesh = plsc.VectorSubcoreMesh(core_axis_name="c", subcore_axis_name="s")

    @functools.partial(
        pl.kernel,
        mesh=mesh,
        out_type=jax.ShapeDtypeStruct((ROWS, F), jnp.float32),
        scratch_types=[
            pltpu.VMEM((nchunks, _CHUNK), jnp.int32),
            pltpu.VMEM((_CHUNK, F), jnp.float32),
            pltpu.VMEM((_CHUNK, F), jnp.float32),
            pltpu.SemaphoreType.DMA,
            pltpu.SemaphoreType.DMA,
        ],
    )
    def scatter(rows_hbm, idx_hbm, out_hbm, idx_v, rows_a, rows_b, sem_a,
                sem_b):
        wid = lax.axis_index("s") * nc + lax.axis_index("c")
        base = wid * rpw
        pltpu.sync_copy(idx_hbm.at[pl.ds(wid * nchunks, nchunks)], idx_v)

        bufs = (rows_a, rows_b)
        sems = (sem_a, sem_b)
        pending = [None, None]
        for j in range(nchunks):
            b = j & 1
            if pending[b] is not None:
                pending[b].wait()
            pltpu.sync_copy(rows_hbm.at[pl.ds(base + j * _CHUNK, _CHUNK)],
                            bufs[b])
            pending[b] = pltpu.async_copy(bufs[b], out_hbm.at[idx_v.at[j]],
                                          sems[b])
        pending[0].wait()
        pending[1].wait()

    return scatter


@functools.cache
def _make_lincopy():
    info = plsc.get_sparse_core_info()
    nc, ns = info.num_cores, info.num_subcores
    nw = nc * ns
    rpw = ROWS // nw
    nchunks = rpw // _CHUNK
    mesh = plsc.VectorSubcoreMesh(core_axis_name="c", subcore_axis_name="s")

    @functools.partial(
        pl.kernel,
        mesh=mesh,
        out_type=jax.ShapeDtypeStruct((ROWS, F), jnp.float32),
        scratch_types=[
            pltpu.VMEM((_CHUNK, F), jnp.float32),
            pltpu.VMEM((_CHUNK, F), jnp.float32),
            pltpu.SemaphoreType.DMA,
            pltpu.SemaphoreType.DMA,
        ],
    )
    def lincopy(rows_hbm, out_hbm, rows_a, rows_b, sem_a, sem_b):
        wid = lax.axis_index("s") * nc + lax.axis_index("c")
        base = wid * rpw
        bufs = (rows_a, rows_b)
        sems = (sem_a, sem_b)
        pending = [None, None]
        for j in range(nchunks):
            b = j & 1
            if pending[b] is not None:
                pending[b].wait()
            off = base + j * _CHUNK
            pltpu.sync_copy(rows_hbm.at[pl.ds(off, _CHUNK)], bufs[b])
            pending[b] = pltpu.async_copy(bufs[b],
                                          out_hbm.at[pl.ds(off, _CHUNK)],
                                          sems[b])
        pending[0].wait()
        pending[1].wait()

    return lincopy


_SBLK = 256


def _sample_body(x_ref, ok_ref):
    nz = (x_ref[...] != 0.0).astype(jnp.float32)
    per_col = jnp.sum(nz, axis=2)
    blockok = (jnp.min(per_col, keepdims=True) > 0.0).astype(jnp.int32)

    @pl.when(pl.program_id(0) == 0)
    def _init():
        ok_ref[...] = blockok

    @pl.when(pl.program_id(0) != 0)
    def _acc():
        ok_ref[...] = jnp.minimum(ok_ref[...], blockok)


_sample_call = pl.pallas_call(
    _sample_body,
    grid=(L // _SBLK,),
    in_specs=[pl.BlockSpec((1, _SBLK, 128), lambda i: (0, i, 0))],
    out_specs=pl.BlockSpec((1, 1), lambda i: (0, 0)),
    out_shape=jax.ShapeDtypeStruct((1, 1), jnp.int32),
)


def _fast_path(x):
    return _make_lincopy()(x.reshape(ROWS, F))


def _slow_path(x):
    colmask = _colmask_call(x)
    dest = _dest_call(colmask)
    return _make_scatter()(x.reshape(ROWS, F),
                           dest.reshape(ROWS // _CHUNK, _CHUNK))


def kernel(inputs):
    ok = _sample_call(inputs)
    out = lax.cond(ok[0, 0] > 0, _fast_path, _slow_path, inputs)
    return out.reshape(B, L, F)
